# baseline (speedup 1.0000x reference)
import jax
import jax.numpy as jnp
from jax.experimental import pallas as pl
from jax.experimental.pallas import tpu as pltpu

_BLOCK_ROWS = 8000


def _body(xp_ref, wb_ref, b_ref, o_ref):
    yp = jnp.dot(
        xp_ref[...].astype(jnp.bfloat16),
        wb_ref[...],
        preferred_element_type=jnp.float32,
    )
    o_ref[...] = yp.reshape(_BLOCK_ROWS, 128) + b_ref[...]


def kernel(x, W, b):
    n, in_dim = x.shape
    emb_dim = W.shape[0]
    rows_per_packed = 128 // in_dim
    xp = x.reshape(n // rows_per_packed, 128)
    wt = W.T.astype(jnp.bfloat16)
    eye8 = jnp.eye(rows_per_packed, dtype=jnp.bfloat16)
    wb = (eye8[:, None, :, None] * wt[None, :, None, :]).reshape(
        128, rows_per_packed * emb_dim
    )
    b2 = b.reshape(1, emb_dim)
    grid = n // _BLOCK_ROWS
    return pl.pallas_call(
        _body,
        grid=(grid,),
        in_specs=[
            pl.BlockSpec((_BLOCK_ROWS // rows_per_packed, 128), lambda i: (i, 0)),
            pl.BlockSpec((128, rows_per_packed * emb_dim), lambda i: (0, 0)),
            pl.BlockSpec((1, emb_dim), lambda i: (0, 0)),
        ],
        out_specs=pl.BlockSpec((_BLOCK_ROWS, emb_dim), lambda i: (i, 0)),
        out_shape=jax.ShapeDtypeStruct((n, emb_dim), jnp.float32),
        compiler_params=pltpu.CompilerParams(
            dimension_semantics=("parallel",),
        ),
    )(xp, wb, b2)


# speedup vs baseline: 1.0850x; 1.0850x over previous
import jax
import jax.numpy as jnp
from jax.experimental import pallas as pl
from jax.experimental.pallas import tpu as pltpu

_BLOCK_ROWS = 32000


def _body(xp_ref, wb_ref, b_ref, o_ref):
    yp = jnp.dot(
        xp_ref[...].astype(jnp.bfloat16),
        wb_ref[...],
        preferred_element_type=jnp.float32,
    )
    o_ref[...] = yp.reshape(_BLOCK_ROWS, 128) + b_ref[...]


def kernel(x, W, b):
    n, in_dim = x.shape
    emb_dim = W.shape[0]
    rows_per_packed = 128 // in_dim
    xp = x.reshape(n // rows_per_packed, 128)
    wt = W.T.astype(jnp.bfloat16)
    eye8 = jnp.eye(rows_per_packed, dtype=jnp.bfloat16)
    wb = (eye8[:, None, :, None] * wt[None, :, None, :]).reshape(
        128, rows_per_packed * emb_dim
    )
    b2 = b.reshape(1, emb_dim)
    grid = n // _BLOCK_ROWS
    return pl.pallas_call(
        _body,
        grid=(grid,),
        in_specs=[
            pl.BlockSpec((_BLOCK_ROWS // rows_per_packed, 128), lambda i: (i, 0)),
            pl.BlockSpec((128, rows_per_packed * emb_dim), lambda i: (0, 0)),
            pl.BlockSpec((1, emb_dim), lambda i: (0, 0)),
        ],
        out_specs=pl.BlockSpec((_BLOCK_ROWS, emb_dim), lambda i: (i, 0)),
        out_shape=jax.ShapeDtypeStruct((n, emb_dim), jnp.float32),
        compiler_params=pltpu.CompilerParams(
            dimension_semantics=("parallel",),
        ),
    )(xp, wb, b2)
